# Initial kernel scaffold; baseline (speedup 1.0000x reference)
import jax
import jax.numpy as jnp
from jax.experimental import pallas as pl


def kernel(x, params):
    raise NotImplementedError("write your pallas kernel here")


# speedup vs baseline: 3.1524x; 3.1524x over previous
import functools

import jax
import jax.numpy as jnp
import numpy as np
from jax.experimental import pallas as pl
from jax.experimental.pallas import tpu as pltpu

_C = 768
_E = 6
_EP = 8
_T = 2048
_ET = (0, 1, 1, 1, 2, 2)
_BLK = 256
_NEG = float("-inf")


def _gelu(v):
    return 0.5 * v * (1.0 + jax.lax.erf(v * (2.0 ** -0.5)))


def _silu(v):
    return v * jax.nn.sigmoid(v)


def _lnorm(h, g, b, eps=1e-5):
    m = jnp.mean(h, axis=-1, keepdims=True)
    var = jnp.mean((h - m) * (h - m), axis=-1, keepdims=True)
    return (h - m) / jnp.sqrt(var + eps) * g + b


def _prep_body(tf8_ref, w1b_ref, rb1_ref, rw2t_ref, ce_ref, w2bar_ref):
    ce_ref[...] = (
        jnp.dot(tf8_ref[...], w1b_ref[...], preferred_element_type=jnp.float32)
        + rb1_ref[...]
    )
    w2bar_ref[...] = jnp.sum(rw2t_ref[...], axis=0, keepdims=True) / float(_E)


def _router_body(x_ref, w1a_ref, ce_ref, w2bar_ref, nw1_ref, nb1_ref,
                 nw2_ref, nb2_ref, norm_ref, bonus_ref, gates_ref):
    xx = x_ref[...]
    xr = jnp.dot(xx, w1a_ref[...], preferred_element_type=jnp.float32)
    w2bar = w2bar_ref[...]
    cols = []
    for e in range(_E):
        ge = _gelu(xr + ce_ref[e:e + 1, :])
        cols.append(jnp.sum(ge * w2bar, axis=1, keepdims=True))
    cols.append(jnp.zeros((xx.shape[0], _EP - _E), jnp.float32))
    logits = jnp.concatenate(cols, axis=1)

    nh = _gelu(jnp.dot(xx, nw1_ref[...], preferred_element_type=jnp.float32)
               + nb1_ref[...])
    nsc = jax.nn.softplus(jax.nn.softplus(
        jnp.dot(nh, nw2_ref[...], preferred_element_type=jnp.float32)
        + nb2_ref[...]))
    noisy = logits + norm_ref[...] * nsc + bonus_ref[...]

    ii = jax.lax.broadcasted_iota(jnp.int32, noisy.shape, 1)
    m1 = jnp.max(noisy, axis=1, keepdims=True)
    i1 = jnp.min(jnp.where(noisy == m1, ii, _EP), axis=1, keepdims=True)
    mk1 = ii == i1
    n2 = jnp.where(mk1, _NEG, noisy)
    m2 = jnp.max(n2, axis=1, keepdims=True)
    i2 = jnp.min(jnp.where(n2 == m2, ii, _EP), axis=1, keepdims=True)
    mk2 = ii == i2
    s2 = jnp.exp(m2 - m1)
    den = 1.0 + s2
    gates_ref[...] = (mk1.astype(jnp.float32)
                      + mk2.astype(jnp.float32) * s2) / den


def _deep_a_body(x_ref, w1, b1, w2, b2, lg, lb, out_ref):
    xx = x_ref[...]
    h = _silu(jnp.dot(xx, w1[...], preferred_element_type=jnp.float32) + b1[...])
    h = jnp.dot(h, w2[...], preferred_element_type=jnp.float32) + b2[...]
    out_ref[...] = _silu(_lnorm(h, lg[...], lb[...]))


def _deep_b_body(acc_ref, x_ref, g_ref, h_ref, w3, b3, ng, nb, out_ref):
    xx = x_ref[...]
    o = jnp.dot(h_ref[...], w3[...], preferred_element_type=jnp.float32) + b3[...]
    y = _lnorm(xx + o, ng[...], nb[...])
    out_ref[...] = acc_ref[...] + g_ref[...] * y


def _wide_body(acc_ref, x_ref, g_ref, w1, b1, lg, lb, w2, b2, ng, nb, out_ref):
    xx = x_ref[...]
    h = _gelu(jnp.dot(xx, w1[...], preferred_element_type=jnp.float32) + b1[...])
    h = _lnorm(h, lg[...], lb[...])
    o = jnp.dot(h, w2[...], preferred_element_type=jnp.float32) + b2[...]
    y = _lnorm(xx + o, ng[...], nb[...])
    out_ref[...] = acc_ref[...] + g_ref[...] * y


def _hybrid_body(acc_ref, x_ref, g_ref, p1w1, p1b1, p1w2, p1b2,
                 p2w1, p2b1, p2w2, p2b2, pw1, pw2, pb, ng, nb, out_ref):
    xx = x_ref[...]
    h1 = _gelu(jnp.dot(xx, p1w1[...], preferred_element_type=jnp.float32)
               + p1b1[...])
    o1 = jnp.dot(h1, p1w2[...], preferred_element_type=jnp.float32) + p1b2[...]
    h2 = _silu(jnp.dot(xx, p2w1[...], preferred_element_type=jnp.float32)
               + p2b1[...])
    o2 = jnp.dot(h2, p2w2[...], preferred_element_type=jnp.float32) + p2b2[...]
    o = (jnp.dot(o1, pw1[...], preferred_element_type=jnp.float32)
         + jnp.dot(o2, pw2[...], preferred_element_type=jnp.float32) + pb[...])
    y = _lnorm(xx + o, ng[...], nb[...])
    out_ref[...] = acc_ref[...] + g_ref[...] * y


def _expert_call(body, acc, xf, gcol, *weights, blk=_BLK):
    specs = [
        pl.BlockSpec((blk, _C), lambda t: (t, 0)),
        pl.BlockSpec((blk, _C), lambda t: (t, 0)),
        pl.BlockSpec((blk, 1), lambda t: (t, 0)),
    ]
    for w in weights:
        specs.append(
            pl.BlockSpec(w.shape, lambda t, n=w.ndim: (0,) * n))
    return pl.pallas_call(
        body,
        grid=(_T // blk,),
        in_specs=specs,
        out_specs=pl.BlockSpec((blk, _C), lambda t: (t, 0)),
        out_shape=jax.ShapeDtypeStruct((_T, _C), jnp.float32),
        input_output_aliases={0: 0},
    )(acc, xf, gcol, *weights)


def _row(v):
    return v.reshape(1, -1)


def kernel(x, params):
    p = params
    xf = x.reshape(_T, _C)
    et = np.array(_ET)

    tf = p["type_emb2"][jnp.array(et, jnp.int32)]
    tf8 = jnp.concatenate([tf, jnp.zeros((_EP - _E, 2 * _C), jnp.float32)], 0)
    w1a = p["route_w1"][:_C]
    w1b = p["route_w1"][_C:]
    rb1 = _row(p["route_b1"])
    rw2t = p["route_w2"].T
    rw2t8 = jnp.concatenate(
        [rw2t, jnp.zeros((_EP - _E, 4 * _C), jnp.float32)], 0)
    b2bar = jnp.mean(p["route_b2"])
    temp = jnp.clip(p["temperature"] * (0.95 ** (_T // 100)), 0.05, 3.0)
    norm = jax.random.normal(jax.random.key(42), (_T, _E), jnp.float32)
    norm_p = jnp.concatenate(
        [temp * norm, jnp.zeros((_T, _EP - _E), jnp.float32)], 1)
    bonus = jnp.full((_EP,), _NEG, jnp.float32)
    bonus = bonus.at[:_E].set(b2bar + 0.3 * (et == 1).astype(jnp.float32))
    bonus = _row(bonus)
    nw1p = jnp.zeros((_C, 128), jnp.float32).at[:, :2 * _E].set(p["noise_w1"])
    nb1p = jnp.zeros((1, 128), jnp.float32).at[0, :2 * _E].set(p["noise_b1"])
    nw2p = jnp.zeros((128, _EP), jnp.float32).at[:2 * _E, :_E].set(p["noise_w2"])
    nb2p = jnp.zeros((1, _EP), jnp.float32).at[0, :_E].set(p["noise_b2"])

    ce, w2bar = pl.pallas_call(
        _prep_body,
        out_shape=[
            jax.ShapeDtypeStruct((_EP, 4 * _C), jnp.float32),
            jax.ShapeDtypeStruct((1, 4 * _C), jnp.float32),
        ],
    )(tf8, w1b, rb1, rw2t8)

    gates = pl.pallas_call(
        _router_body,
        grid=(_T // _BLK,),
        in_specs=[
            pl.BlockSpec((_BLK, _C), lambda t: (t, 0)),
            pl.BlockSpec((_C, 4 * _C), lambda t: (0, 0)),
            pl.BlockSpec((_EP, 4 * _C), lambda t: (0, 0)),
            pl.BlockSpec((1, 4 * _C), lambda t: (0, 0)),
            pl.BlockSpec((_C, 128), lambda t: (0, 0)),
            pl.BlockSpec((1, 128), lambda t: (0, 0)),
            pl.BlockSpec((128, _EP), lambda t: (0, 0)),
            pl.BlockSpec((1, _EP), lambda t: (0, 0)),
            pl.BlockSpec((_BLK, _EP), lambda t: (t, 0)),
            pl.BlockSpec((1, _EP), lambda t: (0, 0)),
        ],
        out_specs=pl.BlockSpec((_BLK, _EP), lambda t: (t, 0)),
        out_shape=jax.ShapeDtypeStruct((_T, _EP), jnp.float32),
    )(xf, w1a, ce, w2bar, nw1p, nb1p, nw2p, nb2p, norm_p, bonus)

    acc = jnp.zeros((_T, _C), jnp.float32)
    ex = p["experts"]
    for e, t in enumerate(_ET):
        q = ex[e]
        gcol = gates[:, e:e + 1]
        if t == 0:
            blk = 128
            hmid = pl.pallas_call(
                _deep_a_body,
                grid=(_T // blk,),
                in_specs=[
                    pl.BlockSpec((blk, _C), lambda t_: (t_, 0)),
                    pl.BlockSpec((_C, 4 * _C), lambda t_: (0, 0)),
                    pl.BlockSpec((1, 4 * _C), lambda t_: (0, 0)),
                    pl.BlockSpec((4 * _C, 4 * _C), lambda t_: (0, 0)),
                    pl.BlockSpec((1, 4 * _C), lambda t_: (0, 0)),
                    pl.BlockSpec((1, 4 * _C), lambda t_: (0, 0)),
                    pl.BlockSpec((1, 4 * _C), lambda t_: (0, 0)),
                ],
                out_specs=pl.BlockSpec((blk, 4 * _C), lambda t_: (t_, 0)),
                out_shape=jax.ShapeDtypeStruct((_T, 4 * _C), jnp.float32),
            )(xf, q["w1"], _row(q["b1"]), q["w2"], _row(q["b2"]),
              _row(q["ln_g"]), _row(q["ln_b"]))
            acc = pl.pallas_call(
                _deep_b_body,
                grid=(_T // _BLK,),
                in_specs=[
                    pl.BlockSpec((_BLK, _C), lambda t_: (t_, 0)),
                    pl.BlockSpec((_BLK, _C), lambda t_: (t_, 0)),
                    pl.BlockSpec((_BLK, 1), lambda t_: (t_, 0)),
                    pl.BlockSpec((_BLK, 4 * _C), lambda t_: (t_, 0)),
                    pl.BlockSpec((4 * _C, _C), lambda t_: (0, 0)),
                    pl.BlockSpec((1, _C), lambda t_: (0, 0)),
                    pl.BlockSpec((1, _C), lambda t_: (0, 0)),
                    pl.BlockSpec((1, _C), lambda t_: (0, 0)),
                ],
                out_specs=pl.BlockSpec((_BLK, _C), lambda t_: (t_, 0)),
                out_shape=jax.ShapeDtypeStruct((_T, _C), jnp.float32),
                input_output_aliases={0: 0},
            )(acc, xf, gcol, hmid, q["w3"], _row(q["b3"]),
              _row(q["ng"]), _row(q["nb"]))
        elif t == 1:
            acc = _expert_call(_wide_body, acc, xf, gcol,
                               q["w1"], _row(q["b1"]),
                               _row(q["ln_g"]), _row(q["ln_b"]),
                               q["w2"], _row(q["b2"]),
                               _row(q["ng"]), _row(q["nb"]))
        else:
            acc = _expert_call(_hybrid_body, acc, xf, gcol,
                               q["p1w1"], _row(q["p1b1"]),
                               q["p1w2"], _row(q["p1b2"]),
                               q["p2w1"], _row(q["p2b1"]),
                               q["p2w2"], _row(q["p2b2"]),
                               q["proj_w"][:_C], q["proj_w"][_C:],
                               _row(q["proj_b"]),
                               _row(q["ng"]), _row(q["nb"]))
    return acc.reshape(1, _T, _C)
